# baseline (speedup 1.0000x reference)
import functools

import jax
import jax.numpy as jnp
from jax import lax
from jax.experimental import pallas as pl
from jax.experimental.pallas import tpu as pltpu
from jax.experimental.pallas import tpu_sc as plsc

NUM_E = 1000000
NUM_R = 1000
D = 64
B = 16384

NC = 2
NS = 16
NW = NC * NS
RPT = B // NW
CH = 128

E_HALF = 1 << 19
E_SHIFT = 19
R_HALF = 1 << 9
R_SHIFT = 9


def _tc_normalize_transpose_pack(xt, half, blk):
    n = xt.shape[1]
    nb = half // blk
    last = pl.cdiv(n, blk) - 1

    def body(x1_ref, x2_ref, o_ref):
        halves = []
        for x_ref in (x1_ref, x2_ref):
            x = x_ref[...]
            ss = jnp.sum(x * x, axis=0, keepdims=True)
            norm = jnp.sqrt(ss)
            halves.append(x / jnp.maximum(norm, 1e-12))
        z = jnp.concatenate(halves, axis=0)
        o_ref[...] = z.T

    return pl.pallas_call(
        body,
        out_shape=jax.ShapeDtypeStruct((half, 2 * D), jnp.float32),
        grid=(nb,),
        in_specs=[
            pl.BlockSpec((D, blk), lambda i: (0, i)),
            pl.BlockSpec((D, blk), lambda i: (0, jnp.minimum(i + nb, last))),
        ],
        out_specs=pl.BlockSpec((blk, 2 * D), lambda i: (i, 0)),
    )(xt, xt)


def _sc_gather(s, r, o, sp, op, e_pack, r_pack):
    mesh = plsc.VectorSubcoreMesh(core_axis_name="c", subcore_axis_name="s")

    nch = RPT // CH
    PITCH = CH + 8

    @functools.partial(
        pl.kernel,
        out_type=jax.ShapeDtypeStruct((5, D, B), jnp.float32),
        mesh=mesh,
        scratch_types=[
            pltpu.VMEM((RPT,), jnp.int32),
            pltpu.VMEM((CH,), jnp.int32),
            pltpu.VMEM((CH,), jnp.int32),
            pltpu.VMEM((CH, 2 * D), jnp.float32),
            pltpu.VMEM((CH, 2 * D), jnp.float32),
            pltpu.VMEM((D, PITCH), jnp.float32),
            pltpu.VMEM((D, PITCH), jnp.float32),
            pltpu.SemaphoreType.DMA,
            pltpu.SemaphoreType.DMA,
            pltpu.SemaphoreType.DMA,
        ],
        compiler_params=pltpu.CompilerParams(
            use_tc_tiling_on_sc=True, needs_layout_passes=False),
    )
    def k(s_h, r_h, o_h, sp_h, op_h, e_h, rel_h, out_h, idx_v, ridx0, ridx1,
          pair0, pair1, rows0, rows1, isem, gsem, osem):
        wid = lax.axis_index("s") * NC + lax.axis_index("c")
        ridx = (ridx0, ridx1)
        pair = (pair0, pair1)
        rows = (rows0, rows1)
        for slot, (idx_h, tbl_h, mask, shift) in enumerate([
            (s_h, e_h, E_HALF - 1, E_SHIFT),
            (r_h, rel_h, R_HALF - 1, R_SHIFT),
            (o_h, e_h, E_HALF - 1, E_SHIFT),
            (sp_h, e_h, E_HALF - 1, E_SHIFT),
            (op_h, e_h, E_HALF - 1, E_SHIFT),
        ]):
            base = wid * RPT
            pltpu.async_copy(idx_h.at[pl.ds(base, RPT)], idx_v, isem).wait()

            def compute_rows(c):
                dst = ridx[c % 2]

                @pl.loop(0, CH, step=16)
                def _rows(g):
                    dst[pl.ds(g, 16)] = idx_v[pl.ds(c * CH + g, 16)] & mask

            def fire(c):
                compute_rows(c)
                return pltpu.async_copy(
                    tbl_h.at[ridx[c % 2]], pair[c % 2], gsem
                )

            gh = fire(0)
            wh = [None, None]
            for c in range(nch):
                gh.wait()
                if c + 1 < nch:
                    gh = fire(c + 1)
                if wh[c % 2] is not None:
                    wh[c % 2].wait()
                pv, rv = pair[c % 2], rows[c % 2]

                @pl.loop(0, CH, step=16)
                def _extract(g):
                    vec = idx_v[pl.ds(c * CH + g, 16)]
                    lane = lax.iota(jnp.int32, 16)
                    for l in range(16):
                        off = ((vec[l] >> shift) & 1) << 6
                        col = jnp.broadcast_to(g + l, (16,))
                        for cc in range(0, D, 16):
                            plsc.store_scatter(
                                rv, [lane + cc, col],
                                pv[g + l, pl.ds(off + cc, 16)],
                            )

                wh[c % 2] = pltpu.async_copy(
                    rv.at[:, pl.ds(0, CH)],
                    out_h.at[slot, :, pl.ds(base + c * CH, CH)],
                    osem,
                )
            for h in wh:
                if h is not None:
                    h.wait()

    return k(s, r, o, sp, op, e_pack, r_pack)


def kernel(s, r, o, sp, op, e_table, r_table):
    s = s.astype(jnp.int32)
    r = r.astype(jnp.int32)
    o = o.astype(jnp.int32)
    sp = sp.astype(jnp.int32)
    op = op.astype(jnp.int32)
    e_pack = _tc_normalize_transpose_pack(
        jnp.swapaxes(e_table, 0, 1), E_HALF, blk=8192)
    r_pack = _tc_normalize_transpose_pack(
        jnp.swapaxes(r_table, 0, 1), R_HALF, blk=512)
    out3 = _sc_gather(s, r, o, sp, op, e_pack, r_pack)
    return jnp.swapaxes(out3, 1, 2)


# speedup vs baseline: 1.1672x; 1.1672x over previous
import functools

import jax
import jax.numpy as jnp
from jax import lax
from jax.experimental import pallas as pl
from jax.experimental.pallas import tpu as pltpu
from jax.experimental.pallas import tpu_sc as plsc

NUM_E = 1000000
NUM_R = 1000
D = 64
B = 16384

NC = 2
NS = 16
NW = NC * NS
RPT = B // NW
CH = 128

E_HALF = 1 << 19
E_SHIFT = 19
R_HALF = 1 << 9
R_SHIFT = 9


def _tc_normalize_transpose_pack(xt, half, blk):
    n = xt.shape[1]
    nb = half // blk
    last = pl.cdiv(n, blk) - 1

    def body(x1_ref, x2_ref, o_ref):
        halves = []
        for x_ref in (x1_ref, x2_ref):
            x = x_ref[...]
            ss = jnp.sum(x * x, axis=0, keepdims=True)
            norm = jnp.sqrt(ss)
            halves.append(x / jnp.maximum(norm, 1e-12))
        z = jnp.concatenate(halves, axis=0)
        o_ref[...] = z.T

    return pl.pallas_call(
        body,
        out_shape=jax.ShapeDtypeStruct((half, 2 * D), jnp.float32),
        grid=(nb,),
        in_specs=[
            pl.BlockSpec((D, blk), lambda i: (0, i)),
            pl.BlockSpec((D, blk), lambda i: (0, jnp.minimum(i + nb, last))),
        ],
        out_specs=pl.BlockSpec((blk, 2 * D), lambda i: (i, 0)),
    )(xt, xt)


def _sc_gather(s, r, o, sp, op, e_pack, r_pack):
    mesh = plsc.VectorSubcoreMesh(core_axis_name="c", subcore_axis_name="s")

    nch = RPT // CH

    @functools.partial(
        pl.kernel,
        out_type=jax.ShapeDtypeStruct((5 * B, D), jnp.float32),
        mesh=mesh,
        scratch_types=[
            pltpu.VMEM((RPT,), jnp.int32),
            pltpu.VMEM((CH,), jnp.int32),
            pltpu.VMEM((CH,), jnp.int32),
            pltpu.VMEM((CH, 2 * D), jnp.float32),
            pltpu.VMEM((CH, 2 * D), jnp.float32),
            pltpu.VMEM((CH, D), jnp.float32),
            pltpu.VMEM((CH, D), jnp.float32),
            pltpu.SemaphoreType.DMA,
            pltpu.SemaphoreType.DMA,
            pltpu.SemaphoreType.DMA,
        ],
        compiler_params=pltpu.CompilerParams(use_tc_tiling_on_sc=True),
    )
    def k(s_h, r_h, o_h, sp_h, op_h, e_h, rel_h, out_h, idx_v, ridx0, ridx1,
          pair0, pair1, rows0, rows1, isem, gsem, osem):
        wid = lax.axis_index("s") * NC + lax.axis_index("c")
        ridx = (ridx0, ridx1)
        pair = (pair0, pair1)
        rows = (rows0, rows1)
        for slot, (idx_h, tbl_h, mask, shift) in enumerate([
            (s_h, e_h, E_HALF - 1, E_SHIFT),
            (r_h, rel_h, R_HALF - 1, R_SHIFT),
            (o_h, e_h, E_HALF - 1, E_SHIFT),
            (sp_h, e_h, E_HALF - 1, E_SHIFT),
            (op_h, e_h, E_HALF - 1, E_SHIFT),
        ]):
            base = wid * RPT
            pltpu.async_copy(idx_h.at[pl.ds(base, RPT)], idx_v, isem).wait()

            def compute_rows(c):
                dst = ridx[c % 2]

                @pl.loop(0, CH, step=16)
                def _rows(g):
                    dst[pl.ds(g, 16)] = idx_v[pl.ds(c * CH + g, 16)] & mask

            def fire(c):
                compute_rows(c)
                return pltpu.async_copy(
                    tbl_h.at[ridx[c % 2]], pair[c % 2], gsem
                )

            gh = fire(0)
            wh = [None, None]
            for c in range(nch):
                gh.wait()
                if c + 1 < nch:
                    gh = fire(c + 1)
                if wh[c % 2] is not None:
                    wh[c % 2].wait()
                pv, rv = pair[c % 2], rows[c % 2]

                @pl.loop(0, CH, step=16)
                def _extract(g):
                    vec = idx_v[pl.ds(c * CH + g, 16)]
                    for l in range(16):
                        off = ((vec[l] >> shift) & 1) << 6
                        for cc in range(0, D, 16):
                            rv[g + l, pl.ds(cc, 16)] = (
                                pv[g + l, pl.ds(off + cc, 16)]
                            )

                wh[c % 2] = pltpu.async_copy(
                    rv, out_h.at[pl.ds(slot * B + base + c * CH, CH)], osem
                )
            for h in wh:
                if h is not None:
                    h.wait()

    return k(s, r, o, sp, op, e_pack, r_pack)


def kernel(s, r, o, sp, op, e_table, r_table):
    s = s.astype(jnp.int32)
    r = r.astype(jnp.int32)
    o = o.astype(jnp.int32)
    sp = sp.astype(jnp.int32)
    op = op.astype(jnp.int32)
    e_pack = _tc_normalize_transpose_pack(
        jnp.swapaxes(e_table, 0, 1), E_HALF, blk=16384)
    r_pack = _tc_normalize_transpose_pack(
        jnp.swapaxes(r_table, 0, 1), R_HALF, blk=512)
    rows = _sc_gather(s, r, o, sp, op, e_pack, r_pack)
    return rows.reshape(5, B, D)


# speedup vs baseline: 1.2053x; 1.0326x over previous
import functools

import jax
import jax.numpy as jnp
from jax import lax
from jax.experimental import pallas as pl
from jax.experimental.pallas import tpu as pltpu
from jax.experimental.pallas import tpu_sc as plsc

NUM_E = 1000000
NUM_R = 1000
D = 64
B = 16384

NC = 2
NS = 16
NW = NC * NS
RPT = B // NW
CH = 128

E_HALF = 1 << 19
E_SHIFT = 19
R_HALF = 1 << 9
R_SHIFT = 9


def _tc_normalize_transpose_pack(xt, half, blk):
    n = xt.shape[1]
    nb = half // blk
    last = pl.cdiv(n, blk) - 1

    def body(x1_ref, x2_ref, o_ref):
        halves = []
        for x_ref in (x1_ref, x2_ref):
            x = x_ref[...]
            ss = jnp.sum(x * x, axis=0, keepdims=True)
            norm = jnp.sqrt(ss)
            halves.append(x / jnp.maximum(norm, 1e-12))
        z = jnp.concatenate(halves, axis=0)
        o_ref[...] = z.T

    return pl.pallas_call(
        body,
        out_shape=jax.ShapeDtypeStruct((half, 2 * D), jnp.float32),
        grid=(nb,),
        in_specs=[
            pl.BlockSpec((D, blk), lambda i: (0, i)),
            pl.BlockSpec((D, blk), lambda i: (0, jnp.minimum(i + nb, last))),
        ],
        out_specs=pl.BlockSpec((blk, 2 * D), lambda i: (i, 0)),
    )(xt, xt)


def _sc_gather(s, r, o, sp, op, e_pack, r_pack):
    mesh = plsc.VectorSubcoreMesh(core_axis_name="c", subcore_axis_name="s")

    nch = RPT // CH
    NBUF = 3

    @functools.partial(
        pl.kernel,
        out_type=jax.ShapeDtypeStruct((5 * B, D), jnp.float32),
        mesh=mesh,
        scratch_types=[
            pltpu.VMEM((5 * RPT,), jnp.int32),
            [pltpu.VMEM((CH,), jnp.int32) for _ in range(NBUF)],
            [pltpu.VMEM((CH, 2 * D), jnp.float32) for _ in range(NBUF)],
            [pltpu.VMEM((CH, D), jnp.float32) for _ in range(2)],
            pltpu.SemaphoreType.DMA,
            [pltpu.SemaphoreType.DMA for _ in range(2)],
            [pltpu.SemaphoreType.DMA for _ in range(2)],
        ],
        compiler_params=pltpu.CompilerParams(use_tc_tiling_on_sc=True),
    )
    def k(s_h, r_h, o_h, sp_h, op_h, e_h, rel_h, out_h, idx_v, ridx,
          pair, rows, isem, gsem, osem):
        wid = lax.axis_index("s") * NC + lax.axis_index("c")
        base = wid * RPT
        idx_hs = (s_h, r_h, o_h, sp_h, op_h)
        for slot in range(5):
            pltpu.async_copy(
                idx_hs[slot].at[pl.ds(base, RPT)],
                idx_v.at[pl.ds(slot * RPT, RPT)], isem)
        for slot in range(5):
            pltpu.make_async_copy(
                idx_hs[slot].at[pl.ds(base, RPT)],
                idx_v.at[pl.ds(slot * RPT, RPT)], isem).wait()

        chunks = []
        for slot, (tbl_h, mask, shift) in enumerate([
            (e_h, E_HALF - 1, E_SHIFT),
            (rel_h, R_HALF - 1, R_SHIFT),
            (e_h, E_HALF - 1, E_SHIFT),
            (e_h, E_HALF - 1, E_SHIFT),
            (e_h, E_HALF - 1, E_SHIFT),
        ]):
            for c in range(nch):
                chunks.append((slot, c, tbl_h, mask, shift))
        ntot = len(chunks)

        def fire(t):
            slot, c, tbl_h, mask, _ = chunks[t]
            dst = ridx[t % NBUF]

            @pl.loop(0, CH, step=16)
            def _rows(g):
                dst[pl.ds(g, 16)] = (
                    idx_v[pl.ds(slot * RPT + c * CH + g, 16)] & mask
                )

            return pltpu.async_copy(tbl_h.at[dst], pair[t % NBUF], gsem[t % 2])

        gh = [fire(0), fire(1)]
        wh = [None, None]
        for t in range(ntot):
            slot, c, tbl_h, mask, shift = chunks[t]
            gh[t % 2].wait()
            if wh[t % 2] is not None:
                wh[t % 2].wait()
            pv, rv = pair[t % NBUF], rows[t % 2]

            @pl.loop(0, CH, step=16)
            def _extract(g):
                vec = idx_v[pl.ds(slot * RPT + c * CH + g, 16)]
                for l in range(16):
                    off = ((vec[l] >> shift) & 1) << 6
                    for cc in range(0, D, 16):
                        rv[g + l, pl.ds(cc, 16)] = (
                            pv[g + l, pl.ds(off + cc, 16)]
                        )

            if t + 2 < ntot:
                gh[t % 2] = fire(t + 2)
            wh[t % 2] = pltpu.async_copy(
                rv, out_h.at[pl.ds(slot * B + base + c * CH, CH)], osem[t % 2]
            )
        for h in wh:
            if h is not None:
                h.wait()

    return k(s, r, o, sp, op, e_pack, r_pack)


def kernel(s, r, o, sp, op, e_table, r_table):
    s = s.astype(jnp.int32)
    r = r.astype(jnp.int32)
    o = o.astype(jnp.int32)
    sp = sp.astype(jnp.int32)
    op = op.astype(jnp.int32)
    e_pack = _tc_normalize_transpose_pack(
        jnp.swapaxes(e_table, 0, 1), E_HALF, blk=16384)
    r_pack = _tc_normalize_transpose_pack(
        jnp.swapaxes(r_table, 0, 1), R_HALF, blk=512)
    rows = _sc_gather(s, r, o, sp, op, e_pack, r_pack)
    return rows.reshape(5, B, D)
